# baseline (speedup 1.0000x reference)
import functools

import jax
import jax.numpy as jnp
from jax import lax
from jax.experimental import pallas as pl
from jax.experimental.pallas import tpu as pltpu
from jax.experimental.pallas import tpu_sc as plsc

BATCH = 16384
EMBED = 64
NROWS = 1000000
NC, NS = 2, 16
NW = NC * NS
WCOLS = 512
NWIN = 1954
WIN_PER = NWIN // NW
WIN_EXTRA = NWIN % NW
NWMAX = WIN_PER + 1
TAIL0 = 1953 * WCOLS
TRASH = BATCH


def _iota16():
    return lax.broadcasted_iota(jnp.int32, (16,), 0)


def _full16(v):
    return jnp.full((16,), v, jnp.int32)


def _sc_body(uids_ref, tTu_ref, tailu_ref,
             gu_ref,
             ids_v, mb_v, wpk_v, buf0, buf1, grps, starts_sm, flags_sm,
             semw0, semw1, gsems):
    t = lax.axis_index("s") * NC + lax.axis_index("c")
    ws = t * WIN_PER + jnp.minimum(t, WIN_EXTRA)
    nw = WIN_PER + (t < WIN_EXTRA).astype(jnp.int32)
    iota = _iota16()

    for r in range(4):
        flags_sm[r] = 0

    HC = WCOLS // 2

    def fetch_half(w, hc, buf, sem, tT_ref, tail_ref):
        @pl.when(w < NWIN - 1)
        def _():
            pltpu.async_copy(
                tT_ref.at[:, pl.ds(w * WCOLS + hc * HC, HC)],
                buf.at[:, pl.ds(hc * HC, HC)], sem)
        if hc == 0:
            @pl.when(w == NWIN - 1)
            def _():
                pltpu.async_copy(tail_ref, buf.at[:, pl.ds(0, 128)], sem)

    def fetch_half_wait(w, hc, buf, sem, tT_ref, tail_ref):
        @pl.when(w < NWIN - 1)
        def _():
            pltpu.make_async_copy(
                tT_ref.at[:, pl.ds(w * WCOLS + hc * HC, HC)],
                buf.at[:, pl.ds(hc * HC, HC)], sem).wait()
        if hc == 0:
            @pl.when(w == NWIN - 1)
            def _():
                pltpu.make_async_copy(
                    tail_ref, buf.at[:, pl.ds(0, 128)], sem).wait()

    def phase(ids_ref, tT_ref, tail_ref, out_ref):
        pltpu.sync_copy(ids_ref, ids_v)

        def s1(g, off):
            idv = ids_v[pl.ds(g * 16, 16)]
            win = lax.shift_right_logical(idv, 9)
            m = (win >= ws) & (win < ws + nw)
            plsc.store_compressed(mb_v.at[pl.ds(off, 16)], g * 16 + iota,
                                  mask=m)
            return off + plsc.all_reduce_population_count(m)[0]

        m_cnt = lax.fori_loop(0, BATCH // 16, s1, 0)

        ngrp_m = (m_cnt + 15) // 16

        def s2(w, woff):
            starts_sm[w] = woff
            col0 = (ws + w) * WCOLS

            def s2g(g, wo):
                mb = mb_v[pl.ds(g * 16, 16)] & (BATCH - 1)
                valid = (g * 16 + iota) < m_cnt
                j = plsc.load_gather(ids_v, [mb])
                m = valid & (lax.shift_right_logical(j, 9) == ws + w)
                pack = mb | ((j - col0) << 14)
                plsc.store_compressed(wpk_v.at[pl.ds(wo, 16)], pack, mask=m)
                return wo + plsc.all_reduce_population_count(m)[0]

            return lax.fori_loop(0, ngrp_m, s2g, woff)

        wtot = lax.fori_loop(0, nw, s2, 0)
        starts_sm[nw] = wtot

        for wi0 in range(2):
            @pl.when(wi0 < nw)
            def _(wi0=wi0):
                bufp, semsp = (buf0, semw0) if wi0 == 0 else (buf1, semw1)
                for hc in range(2):
                    fetch_half(ws + wi0, hc, bufp, semsp[hc], tT_ref, tail_ref)

        def group(off, e, buf, grpbuf, gsem, fl):
            @pl.when(flags_sm[fl] == 1)
            def _():
                pltpu.make_async_copy(
                    grpbuf, out_ref.at[pl.ds(0, 16)], gsem).wait()
            wpk = wpk_v[pl.ds(off, 16)]
            wj = lax.shift_right_logical(wpk, 14) & (WCOLS - 1)
            wb = wpk & (BATCH - 1)
            bt = jnp.where(iota < e - off, wb, TRASH + iota)
            for c in range(EMBED):
                vals = plsc.load_gather(buf, [_full16(c), wj])
                plsc.store_scatter(grpbuf, [iota, _full16(c)], vals)
            pltpu.async_copy(grpbuf, out_ref.at[bt], gsem)
            flags_sm[fl] = 1

        def window(wi, buf, mysems):
            w = ws + wi
            for hc in range(2):
                fetch_half_wait(w, hc, buf, mysems[hc], tT_ref, tail_ref)

            s = starts_sm[wi]
            e = starts_sm[wi + 1]

            def quad(q, _):
                for r in range(4):
                    off = s + q * 64 + r * 16

                    @pl.when(off < e)
                    def _():
                        group(off, e, buf, grps[r], gsems[r], r)
                return 0

            lax.fori_loop(0, (e - s + 63) // 64, quad, 0)

            @pl.when(wi + 2 < nw)
            def _():
                for hc in range(2):
                    fetch_half(w + 2, hc, buf, mysems[hc], tT_ref, tail_ref)

        def wpair(p, _):
            for par in range(2):
                wi = p * 2 + par

                @pl.when(wi < nw)
                def _():
                    if par == 0:
                        window(wi, buf0, semw0)
                    else:
                        window(wi, buf1, semw1)
            return 0

        lax.fori_loop(0, (NWMAX + 1) // 2, wpair, 0)

        for r in range(4):
            @pl.when(flags_sm[r] == 1)
            def _():
                pltpu.make_async_copy(
                    grps[r], out_ref.at[pl.ds(0, 16)], gsems[r]).wait()
            flags_sm[r] = 0

    phase(uids_ref, tTu_ref, tailu_ref, gu_ref)


def _sc_body_wrap(uids_ref, tTu_ref, tailu_ref,
                  gu_ref,
                  ids_v, mb_v, wpk_v, buf0, buf1, g0, g1, g2, g3,
                  starts_sm, flags_sm, wa0, wa1, wa2, wa3, wb0, wb1, wb2, wb3,
                  s0, s1, s2, s3):
    _sc_body(uids_ref, tTu_ref, tailu_ref,
             gu_ref,
             ids_v, mb_v, wpk_v, buf0, buf1, (g0, g1, g2, g3),
             starts_sm, flags_sm, (wa0, wa1, wa2, wa3), (wb0, wb1, wb2, wb3),
             (s0, s1, s2, s3))


def _sc_gather(user_ids, user_table):
    mesh = plsc.VectorSubcoreMesh(core_axis_name="c", subcore_axis_name="s",
                                  num_cores=NC, num_subcores=NS)
    f = pl.kernel(
        _sc_body_wrap,
        out_type=jax.ShapeDtypeStruct((BATCH + 128, 128), jnp.float32),
        mesh=mesh,
        compiler_params=pltpu.CompilerParams(needs_layout_passes=False),
        scratch_types=[
            pltpu.VMEM((BATCH,), jnp.int32),
            pltpu.VMEM((BATCH + 64,), jnp.int32),
            pltpu.VMEM((BATCH + 64,), jnp.int32),
            pltpu.VMEM((EMBED, WCOLS), jnp.float32),
            pltpu.VMEM((EMBED, WCOLS), jnp.float32),
            pltpu.VMEM((16, 128), jnp.float32),
            pltpu.VMEM((16, 128), jnp.float32),
            pltpu.VMEM((16, 128), jnp.float32),
            pltpu.VMEM((16, 128), jnp.float32),
            pltpu.SMEM((NWMAX + 1,), jnp.int32),
            pltpu.SMEM((4,), jnp.int32),
        ] + [pltpu.SemaphoreType.DMA] * 12,
    )
    tTu = user_table.T
    tailu = jnp.zeros((EMBED, 128), jnp.float32).at[:, :EMBED].set(
        tTu[:, TAIL0:])
    return f(user_ids.astype(jnp.int32), tTu, tailu)


def _sc_pair_body(idx_ref, t2_ref, out_ref, idx_v, rows_v, sem):
    wid = lax.axis_index("s") * NC + lax.axis_index("c")
    base = wid * (BATCH // NW)
    pltpu.sync_copy(idx_ref.at[pl.ds(wid * 4, 4)], idx_v)
    descs = []
    for j in range(4):
        descs.append(pltpu.async_copy(
            t2_ref.at[idx_v.at[j]], rows_v.at[pl.ds(j * 128, 128)], sem))
    for d in descs:
        d.wait()
    pltpu.sync_copy(rows_v, out_ref.at[pl.ds(base, BATCH // NW)])


def _sc_pair_gather(idx2d, t2):
    mesh = plsc.VectorSubcoreMesh(core_axis_name="c", subcore_axis_name="s",
                                  num_cores=NC, num_subcores=NS)
    f = pl.kernel(
        _sc_pair_body,
        out_type=jax.ShapeDtypeStruct((BATCH, 128), jnp.float32),
        mesh=mesh,
        scratch_types=[
            pltpu.VMEM((4, 128), jnp.int32),
            pltpu.VMEM((BATCH // NW, 128), jnp.float32),
            pltpu.SemaphoreType.DMA,
        ],
    )
    return f(idx2d, t2)


def _mlp_body(ue_ref, ie_ref, par_ref, w1u_ref, w1i_ref, b1_ref, w2_ref,
              b2_ref, w3_ref, b3_ref, out_ref):
    ue = ue_ref[...][:, :EMBED]
    gi = ie_ref[...]
    ie = jnp.where(par_ref[...] > 0.5, gi[:, EMBED:], gi[:, :EMBED])
    h = jnp.dot(ue, w1u_ref[...], preferred_element_type=jnp.float32,
                precision=lax.Precision.HIGHEST)
    h += jnp.dot(ie, w1i_ref[...], preferred_element_type=jnp.float32,
                 precision=lax.Precision.HIGHEST)
    h = jnp.maximum(h + b1_ref[...], 0.0)
    h = jnp.dot(h, w2_ref[...], preferred_element_type=jnp.float32,
                precision=lax.Precision.HIGHEST)
    h = jnp.maximum(h + b2_ref[...], 0.0)
    z = jnp.sum(h * w3_ref[...], axis=1, keepdims=True) + b3_ref[0, 0]
    out_ref[...] = 1.0 / (1.0 + jnp.exp(-z))


def _mlp(gu, gi, par, W1u, W1i, b1, W2, b2, w3row, b3, block_b=2048):
    grid = BATCH // block_b
    full = lambda r, c: pl.BlockSpec((r, c), lambda i: (0, 0))
    return pl.pallas_call(
        _mlp_body,
        grid=(grid,),
        in_specs=[
            pl.BlockSpec((block_b, 128), lambda i: (i, 0)),
            pl.BlockSpec((block_b, 128), lambda i: (i, 0)),
            pl.BlockSpec((block_b, 1), lambda i: (i, 0)),
            full(EMBED, 128), full(EMBED, 128), full(1, 128),
            full(128, EMBED), full(1, EMBED),
            full(1, EMBED), full(1, 1),
        ],
        out_specs=pl.BlockSpec((block_b, 1), lambda i: (i, 0)),
        out_shape=jax.ShapeDtypeStruct((BATCH, 1), jnp.float32),
    )(gu, gi, par, W1u, W1i, b1, W2, b2, w3row, b3)


def kernel(user_ids, item_ids, user_table, item_table, W1, b1, W2, b2, W3, b3):
    gu = _sc_gather(user_ids, user_table)
    iid = item_ids.astype(jnp.int32)
    t2i = jnp.concatenate([item_table[0::2], item_table[1::2]], axis=1)
    idx2d = (iid >> 1).reshape(BATCH // 128, 128)
    par = (iid & 1).astype(jnp.float32).reshape(BATCH, 1)
    gi = _sc_pair_gather(idx2d, t2i)
    W1u, W1i = W1[:EMBED], W1[EMBED:]
    out = _mlp(gu, gi, par, W1u, W1i, b1.reshape(1, -1), W2,
               b2.reshape(1, -1), W3.reshape(1, -1), b3.reshape(1, 1))
    return out.reshape(BATCH)


# speedup vs baseline: 14.9605x; 14.9605x over previous
import functools

import jax
import jax.numpy as jnp
from jax import lax
from jax.experimental import pallas as pl
from jax.experimental.pallas import tpu as pltpu
from jax.experimental.pallas import tpu_sc as plsc

BATCH = 16384
EMBED = 64
NROWS = 1000000
NC, NS = 2, 16
NW = NC * NS
WCOLS = 512
NWIN = 1954
WIN_PER = NWIN // NW
WIN_EXTRA = NWIN % NW
NWMAX = WIN_PER + 1
TAIL0 = 1953 * WCOLS
TRASH = BATCH


def _iota16():
    return lax.broadcasted_iota(jnp.int32, (16,), 0)


def _full16(v):
    return jnp.full((16,), v, jnp.int32)


def _sc_body(uids_ref, iids_ref, tTu_ref, tTi_ref, tailu_ref, taili_ref,
             gu_ref, gi_ref,
             ids_v, mb_v, wpk_v, buf0, buf1, grps, starts_sm, flags_sm,
             semw0, semw1, gsems):
    t = lax.axis_index("s") * NC + lax.axis_index("c")
    ws = t * WIN_PER + jnp.minimum(t, WIN_EXTRA)
    nw = WIN_PER + (t < WIN_EXTRA).astype(jnp.int32)
    iota = _iota16()

    for r in range(4):
        flags_sm[r] = 0

    HC = WCOLS // 2

    def fetch_half(w, hc, buf, sem, tT_ref, tail_ref):
        @pl.when(w < NWIN - 1)
        def _():
            pltpu.async_copy(
                tT_ref.at[:, pl.ds(w * WCOLS + hc * HC, HC)],
                buf.at[:, pl.ds(hc * HC, HC)], sem)
        if hc == 0:
            @pl.when(w == NWIN - 1)
            def _():
                pltpu.async_copy(tail_ref, buf.at[:, pl.ds(0, 128)], sem)

    def fetch_half_wait(w, hc, buf, sem, tT_ref, tail_ref):
        @pl.when(w < NWIN - 1)
        def _():
            pltpu.make_async_copy(
                tT_ref.at[:, pl.ds(w * WCOLS + hc * HC, HC)],
                buf.at[:, pl.ds(hc * HC, HC)], sem).wait()
        if hc == 0:
            @pl.when(w == NWIN - 1)
            def _():
                pltpu.make_async_copy(
                    tail_ref, buf.at[:, pl.ds(0, 128)], sem).wait()

    def phase(ids_ref, tT_ref, tail_ref, out_ref):
        pltpu.sync_copy(ids_ref, ids_v)

        def s1(g, off):
            idv = ids_v[pl.ds(g * 16, 16)]
            win = lax.shift_right_logical(idv, 9)
            m = (win >= ws) & (win < ws + nw)
            plsc.store_compressed(mb_v.at[pl.ds(off, 16)], g * 16 + iota,
                                  mask=m)
            return off + plsc.all_reduce_population_count(m)[0]

        m_cnt = lax.fori_loop(0, BATCH // 16, s1, 0)

        ngrp_m = (m_cnt + 15) // 16

        def s2(w, woff):
            starts_sm[w] = woff
            col0 = (ws + w) * WCOLS

            def s2g(g, wo):
                mb = mb_v[pl.ds(g * 16, 16)] & (BATCH - 1)
                valid = (g * 16 + iota) < m_cnt
                j = plsc.load_gather(ids_v, [mb])
                m = valid & (lax.shift_right_logical(j, 9) == ws + w)
                pack = mb | ((j - col0) << 14)
                plsc.store_compressed(wpk_v.at[pl.ds(wo, 16)], pack, mask=m)
                return wo + plsc.all_reduce_population_count(m)[0]

            return lax.fori_loop(0, ngrp_m, s2g, woff)

        wtot = lax.fori_loop(0, nw, s2, 0)
        starts_sm[nw] = wtot

        for wi0 in range(2):
            @pl.when(wi0 < nw)
            def _(wi0=wi0):
                bufp, semsp = (buf0, semw0) if wi0 == 0 else (buf1, semw1)
                for hc in range(2):
                    fetch_half(ws + wi0, hc, bufp, semsp[hc], tT_ref, tail_ref)

        def group(off, e, buf, grpbuf, gsem, fl):
            @pl.when(flags_sm[fl] == 1)
            def _():
                pltpu.make_async_copy(
                    grpbuf, out_ref.at[pl.ds(0, 16)], gsem).wait()
            wpk = wpk_v[pl.ds(off, 16)]
            wj = lax.shift_right_logical(wpk, 14) & (WCOLS - 1)
            wb = wpk & (BATCH - 1)
            bt = jnp.where(iota < e - off, wb, TRASH + iota)
            for c in range(EMBED):
                vals = plsc.load_gather(buf, [_full16(c), wj])
                plsc.store_scatter(grpbuf, [iota, _full16(c)], vals)
            pltpu.async_copy(grpbuf, out_ref.at[bt], gsem)
            flags_sm[fl] = 1

        def window(wi, buf, mysems):
            w = ws + wi
            for hc in range(2):
                fetch_half_wait(w, hc, buf, mysems[hc], tT_ref, tail_ref)

            s = starts_sm[wi]
            e = starts_sm[wi + 1]

            def quad(q, _):
                for r in range(4):
                    off = s + q * 64 + r * 16

                    @pl.when(off < e)
                    def _():
                        group(off, e, buf, grps[r], gsems[r], r)
                return 0

            lax.fori_loop(0, (e - s + 63) // 64, quad, 0)

            @pl.when(wi + 2 < nw)
            def _():
                for hc in range(2):
                    fetch_half(w + 2, hc, buf, mysems[hc], tT_ref, tail_ref)

        def wpair(p, _):
            for par in range(2):
                wi = p * 2 + par

                @pl.when(wi < nw)
                def _():
                    if par == 0:
                        window(wi, buf0, semw0)
                    else:
                        window(wi, buf1, semw1)
            return 0

        lax.fori_loop(0, (NWMAX + 1) // 2, wpair, 0)

        for r in range(4):
            @pl.when(flags_sm[r] == 1)
            def _():
                pltpu.make_async_copy(
                    grps[r], out_ref.at[pl.ds(0, 16)], gsems[r]).wait()
            flags_sm[r] = 0

    phase(uids_ref, tTu_ref, tailu_ref, gu_ref)
    phase(iids_ref, tTi_ref, taili_ref, gi_ref)


def _sc_body_wrap(uids_ref, iids_ref, tTu_ref, tTi_ref, tailu_ref, taili_ref,
                  gu_ref, gi_ref,
                  ids_v, mb_v, wpk_v, buf0, buf1, g0, g1, g2, g3,
                  starts_sm, flags_sm, wa0, wa1, wa2, wa3, wb0, wb1, wb2, wb3,
                  s0, s1, s2, s3):
    _sc_body(uids_ref, iids_ref, tTu_ref, tTi_ref, tailu_ref, taili_ref,
             gu_ref, gi_ref,
             ids_v, mb_v, wpk_v, buf0, buf1, (g0, g1, g2, g3),
             starts_sm, flags_sm, (wa0, wa1, wa2, wa3), (wb0, wb1, wb2, wb3),
             (s0, s1, s2, s3))


def _sc_gather(user_ids, item_ids, user_table, item_table):
    mesh = plsc.VectorSubcoreMesh(core_axis_name="c", subcore_axis_name="s",
                                  num_cores=NC, num_subcores=NS)
    f = pl.kernel(
        _sc_body_wrap,
        out_type=(jax.ShapeDtypeStruct((BATCH + 128, 128), jnp.float32),
                  jax.ShapeDtypeStruct((BATCH + 128, 128), jnp.float32)),
        mesh=mesh,
        compiler_params=pltpu.CompilerParams(needs_layout_passes=False),
        scratch_types=[
            pltpu.VMEM((BATCH,), jnp.int32),
            pltpu.VMEM((BATCH + 64,), jnp.int32),
            pltpu.VMEM((BATCH + 64,), jnp.int32),
            pltpu.VMEM((EMBED, WCOLS), jnp.float32),
            pltpu.VMEM((EMBED, WCOLS), jnp.float32),
            pltpu.VMEM((16, 128), jnp.float32),
            pltpu.VMEM((16, 128), jnp.float32),
            pltpu.VMEM((16, 128), jnp.float32),
            pltpu.VMEM((16, 128), jnp.float32),
            pltpu.SMEM((NWMAX + 1,), jnp.int32),
            pltpu.SMEM((4,), jnp.int32),
        ] + [pltpu.SemaphoreType.DMA] * 12,
    )
    tTu = user_table.T
    tTi = item_table.T
    tailu = jnp.zeros((EMBED, 128), jnp.float32).at[:, :EMBED].set(
        tTu[:, TAIL0:])
    taili = jnp.zeros((EMBED, 128), jnp.float32).at[:, :EMBED].set(
        tTi[:, TAIL0:])
    return f(user_ids.astype(jnp.int32), item_ids.astype(jnp.int32),
             tTu, tTi, tailu, taili)


def _mlp_body(ue_ref, ie_ref, par_ref, w1u_ref, w1i_ref, b1_ref, w2_ref,
              b2_ref, w3_ref, b3_ref, out_ref):
    ue = ue_ref[...][:, :EMBED]
    gi = ie_ref[...]
    ie = jnp.where(par_ref[...] > 0.5, gi[:, EMBED:], gi[:, :EMBED])
    h = jnp.dot(ue, w1u_ref[...], preferred_element_type=jnp.float32,
                precision=lax.Precision.HIGHEST)
    h += jnp.dot(ie, w1i_ref[...], preferred_element_type=jnp.float32,
                 precision=lax.Precision.HIGHEST)
    h = jnp.maximum(h + b1_ref[...], 0.0)
    h = jnp.dot(h, w2_ref[...], preferred_element_type=jnp.float32,
                precision=lax.Precision.HIGHEST)
    h = jnp.maximum(h + b2_ref[...], 0.0)
    z = jnp.sum(h * w3_ref[...], axis=1, keepdims=True) + b3_ref[0, 0]
    out_ref[...] = 1.0 / (1.0 + jnp.exp(-z))


def _mlp(gu, gi, par, W1u, W1i, b1, W2, b2, w3row, b3, block_b=2048):
    grid = BATCH // block_b
    full = lambda r, c: pl.BlockSpec((r, c), lambda i: (0, 0))
    return pl.pallas_call(
        _mlp_body,
        grid=(grid,),
        in_specs=[
            pl.BlockSpec((block_b, 128), lambda i: (i, 0)),
            pl.BlockSpec((block_b, 128), lambda i: (i, 0)),
            pl.BlockSpec((block_b, 1), lambda i: (i, 0)),
            full(EMBED, 128), full(EMBED, 128), full(1, 128),
            full(128, EMBED), full(1, EMBED),
            full(1, EMBED), full(1, 1),
        ],
        out_specs=pl.BlockSpec((block_b, 1), lambda i: (i, 0)),
        out_shape=jax.ShapeDtypeStruct((BATCH, 1), jnp.float32),
    )(gu, gi, par, W1u, W1i, b1, W2, b2, w3row, b3)


def kernel(user_ids, item_ids, user_table, item_table, W1, b1, W2, b2, W3, b3):
    gu, gi = _sc_gather(user_ids, item_ids, user_table, item_table)
    par = jnp.zeros((BATCH, 1), jnp.float32)
    W1u, W1i = W1[:EMBED], W1[EMBED:]
    out = _mlp(gu, gi, par, W1u, W1i, b1.reshape(1, -1), W2,
               b2.reshape(1, -1), W3.reshape(1, -1), b3.reshape(1, 1))
    return out.reshape(BATCH)


# speedup vs baseline: 15.1574x; 1.0132x over previous
import functools

import jax
import jax.numpy as jnp
from jax import lax
from jax.experimental import pallas as pl
from jax.experimental.pallas import tpu as pltpu
from jax.experimental.pallas import tpu_sc as plsc

BATCH = 16384
EMBED = 64
NROWS = 1000000
NC, NS = 2, 16
NW = NC * NS
WCOLS = 512
NWIN = 1954
WIN_PER = NWIN // NW
WIN_EXTRA = NWIN % NW
NWMAX = WIN_PER + 1
TAIL0 = 1953 * WCOLS
TRASH = BATCH


def _iota16():
    return lax.broadcasted_iota(jnp.int32, (16,), 0)


def _full16(v):
    return jnp.full((16,), v, jnp.int32)


def _sc_body(uids_ref, iids_ref, tTu_ref, tTi_ref, tailu_ref, taili_ref,
             gu_ref, gi_ref,
             ids_v, mb_v, wpk_v, buf0, buf1, grps, starts_sm, flags_sm,
             semw0, semw1, gsems):
    t = lax.axis_index("s") * NC + lax.axis_index("c")
    ws = t * WIN_PER + jnp.minimum(t, WIN_EXTRA)
    nw = WIN_PER + (t < WIN_EXTRA).astype(jnp.int32)
    iota = _iota16()

    for r in range(4):
        flags_sm[r] = 0

    HC = WCOLS // 2

    def fetch_half(w, hc, buf, sem, tT_ref, tail_ref):
        @pl.when(w < NWIN - 1)
        def _():
            pltpu.async_copy(
                tT_ref.at[:, pl.ds(w * WCOLS + hc * HC, HC)],
                buf.at[:, pl.ds(hc * HC, HC)], sem)
        if hc == 0:
            @pl.when(w == NWIN - 1)
            def _():
                pltpu.async_copy(tail_ref, buf.at[:, pl.ds(0, 128)], sem)

    def fetch_half_wait(w, hc, buf, sem, tT_ref, tail_ref):
        @pl.when(w < NWIN - 1)
        def _():
            pltpu.make_async_copy(
                tT_ref.at[:, pl.ds(w * WCOLS + hc * HC, HC)],
                buf.at[:, pl.ds(hc * HC, HC)], sem).wait()
        if hc == 0:
            @pl.when(w == NWIN - 1)
            def _():
                pltpu.make_async_copy(
                    tail_ref, buf.at[:, pl.ds(0, 128)], sem).wait()

    def phase(ids_ref, tT_ref, tail_ref, out_ref):
        pltpu.sync_copy(ids_ref, ids_v)

        def s1(g, off):
            idv = ids_v[pl.ds(g * 16, 16)]
            win = lax.shift_right_logical(idv, 9)
            m = (win >= ws) & (win < ws + nw)
            plsc.store_compressed(mb_v.at[pl.ds(off, 16)], g * 16 + iota,
                                  mask=m)
            return off + plsc.all_reduce_population_count(m)[0]

        m_cnt = lax.fori_loop(0, BATCH // 16, s1, 0)

        ngrp_m = (m_cnt + 15) // 16

        def s2(w, woff):
            starts_sm[w] = woff
            col0 = (ws + w) * WCOLS

            def s2g(g, wo):
                mb = mb_v[pl.ds(g * 16, 16)] & (BATCH - 1)
                valid = (g * 16 + iota) < m_cnt
                j = plsc.load_gather(ids_v, [mb])
                m = valid & (lax.shift_right_logical(j, 9) == ws + w)
                pack = mb | ((j - col0) << 14)
                plsc.store_compressed(wpk_v.at[pl.ds(wo, 16)], pack, mask=m)
                return wo + plsc.all_reduce_population_count(m)[0]

            return lax.fori_loop(0, ngrp_m, s2g, woff)

        wtot = lax.fori_loop(0, nw, s2, 0)
        starts_sm[nw] = wtot

        for wi0 in range(2):
            @pl.when(wi0 < nw)
            def _(wi0=wi0):
                bufp, semsp = (buf0, semw0) if wi0 == 0 else (buf1, semw1)
                for hc in range(2):
                    fetch_half(ws + wi0, hc, bufp, semsp[hc], tT_ref, tail_ref)

        def group(off, e, buf, grpbuf, gsem, fl):
            @pl.when(flags_sm[fl] == 1)
            def _():
                pltpu.make_async_copy(
                    grpbuf, out_ref.at[pl.ds(0, 16)], gsem).wait()
            wpk = wpk_v[pl.ds(off, 16)]
            wj = lax.shift_right_logical(wpk, 14) & (WCOLS - 1)
            wb = wpk & (BATCH - 1)
            bt = jnp.where(iota < e - off, wb, TRASH + iota)
            for c in range(EMBED):
                vals = plsc.load_gather(buf, [_full16(c), wj])
                plsc.store_scatter(grpbuf, [iota, _full16(c)], vals)
            pltpu.async_copy(grpbuf, out_ref.at[bt], gsem)
            flags_sm[fl] = 1

        def window(wi, buf, mysems):
            w = ws + wi
            for hc in range(2):
                fetch_half_wait(w, hc, buf, mysems[hc], tT_ref, tail_ref)

            s = starts_sm[wi]
            e = starts_sm[wi + 1]

            def quad(q, _):
                for r in range(4):
                    off = s + q * 64 + r * 16

                    @pl.when(off < e)
                    def _():
                        group(off, e, buf, grps[r], gsems[r], r)
                return 0

            lax.fori_loop(0, (e - s + 63) // 64, quad, 0)

            @pl.when(wi + 2 < nw)
            def _():
                for hc in range(2):
                    fetch_half(w + 2, hc, buf, mysems[hc], tT_ref, tail_ref)

        def wpair(p, _):
            for par in range(2):
                wi = p * 2 + par

                @pl.when(wi < nw)
                def _():
                    if par == 0:
                        window(wi, buf0, semw0)
                    else:
                        window(wi, buf1, semw1)
            return 0

        lax.fori_loop(0, (NWMAX + 1) // 2, wpair, 0)

        for r in range(4):
            @pl.when(flags_sm[r] == 1)
            def _():
                pltpu.make_async_copy(
                    grps[r], out_ref.at[pl.ds(0, 16)], gsems[r]).wait()
            flags_sm[r] = 0

    phase(uids_ref, tTu_ref, tailu_ref, gu_ref)
    phase(iids_ref, tTi_ref, taili_ref, gi_ref)


def _sc_body_wrap(uids_ref, iids_ref, tTu_ref, tTi_ref, tailu_ref, taili_ref,
                  gu_ref, gi_ref,
                  ids_v, mb_v, wpk_v, buf0, buf1, g0, g1, g2, g3,
                  starts_sm, flags_sm, wa0, wa1, wa2, wa3, wb0, wb1, wb2, wb3,
                  s0, s1, s2, s3):
    _sc_body(uids_ref, iids_ref, tTu_ref, tTi_ref, tailu_ref, taili_ref,
             gu_ref, gi_ref,
             ids_v, mb_v, wpk_v, buf0, buf1, (g0, g1, g2, g3),
             starts_sm, flags_sm, (wa0, wa1, wa2, wa3), (wb0, wb1, wb2, wb3),
             (s0, s1, s2, s3))


def _sc_gather(user_ids, item_ids, user_table, item_table):
    mesh = plsc.VectorSubcoreMesh(core_axis_name="c", subcore_axis_name="s",
                                  num_cores=NC, num_subcores=NS)
    f = pl.kernel(
        _sc_body_wrap,
        out_type=(jax.ShapeDtypeStruct((BATCH + 128, 128), jnp.float32),
                  jax.ShapeDtypeStruct((BATCH + 128, 128), jnp.float32)),
        mesh=mesh,
        compiler_params=pltpu.CompilerParams(needs_layout_passes=False),
        scratch_types=[
            pltpu.VMEM((BATCH,), jnp.int32),
            pltpu.VMEM((BATCH + 64,), jnp.int32),
            pltpu.VMEM((BATCH + 64,), jnp.int32),
            pltpu.VMEM((EMBED, WCOLS), jnp.float32),
            pltpu.VMEM((EMBED, WCOLS), jnp.float32),
            pltpu.VMEM((16, 128), jnp.float32),
            pltpu.VMEM((16, 128), jnp.float32),
            pltpu.VMEM((16, 128), jnp.float32),
            pltpu.VMEM((16, 128), jnp.float32),
            pltpu.SMEM((NWMAX + 1,), jnp.int32),
            pltpu.SMEM((4,), jnp.int32),
        ] + [pltpu.SemaphoreType.DMA] * 12,
    )
    tTu = user_table.T
    tTi = item_table.T
    tailu = jnp.zeros((EMBED, 128), jnp.float32).at[:, :EMBED].set(
        tTu[:, TAIL0:])
    taili = jnp.zeros((EMBED, 128), jnp.float32).at[:, :EMBED].set(
        tTi[:, TAIL0:])
    return f(user_ids.astype(jnp.int32), item_ids.astype(jnp.int32),
             tTu, tTi, tailu, taili)


def _mlp_body(ue_ref, ie_ref, w1u_ref, w1i_ref, b1_ref, w2_ref,
              b2_ref, w3_ref, b3_ref, out_ref):
    ue = ue_ref[...][:, :EMBED]
    ie = ie_ref[...][:, :EMBED]
    h = jnp.dot(ue, w1u_ref[...], preferred_element_type=jnp.float32,
                precision=lax.Precision.HIGHEST)
    h += jnp.dot(ie, w1i_ref[...], preferred_element_type=jnp.float32,
                 precision=lax.Precision.HIGHEST)
    h = jnp.maximum(h + b1_ref[...], 0.0)
    h = jnp.dot(h, w2_ref[...], preferred_element_type=jnp.float32,
                precision=lax.Precision.HIGHEST)
    h = jnp.maximum(h + b2_ref[...], 0.0)
    z = jnp.sum(h * w3_ref[...], axis=1, keepdims=True) + b3_ref[0, 0]
    out_ref[...] = 1.0 / (1.0 + jnp.exp(-z))


def _mlp(gu, gi, W1u, W1i, b1, W2, b2, w3row, b3, block_b=2048):
    grid = BATCH // block_b
    full = lambda r, c: pl.BlockSpec((r, c), lambda i: (0, 0))
    return pl.pallas_call(
        _mlp_body,
        grid=(grid,),
        in_specs=[
            pl.BlockSpec((block_b, 128), lambda i: (i, 0)),
            pl.BlockSpec((block_b, 128), lambda i: (i, 0)),
            full(EMBED, 128), full(EMBED, 128), full(1, 128),
            full(128, EMBED), full(1, EMBED),
            full(1, EMBED), full(1, 1),
        ],
        out_specs=pl.BlockSpec((block_b, 1), lambda i: (i, 0)),
        out_shape=jax.ShapeDtypeStruct((BATCH, 1), jnp.float32),
    )(gu, gi, W1u, W1i, b1, W2, b2, w3row, b3)


def kernel(user_ids, item_ids, user_table, item_table, W1, b1, W2, b2, W3, b3):
    gu, gi = _sc_gather(user_ids, item_ids, user_table, item_table)
    W1u, W1i = W1[:EMBED], W1[EMBED:]
    out = _mlp(gu, gi, W1u, W1i, b1.reshape(1, -1), W2,
               b2.reshape(1, -1), W3.reshape(1, -1), b3.reshape(1, 1))
    return out.reshape(BATCH)
